# baseline (speedup 1.0000x reference)
import functools

import jax
import jax.numpy as jnp
from jax.experimental import pallas as pl
from jax.experimental.pallas import tpu as pltpu

ANCHOR_INTERVAL = 16
_B, _H, _S, _D = 4, 16, 4096, 128
_A = _S // ANCHOR_INTERVAL
_G = _B * _H
_GB = 4


def _tc_body(k4, v4, ko, vo):
    ko[...] = k4[...]
    vo[...] = v4[...]


def kernel(k, v):
    k4 = k.reshape(_G, _A, ANCHOR_INTERVAL, 1, _D)
    v4 = v.reshape(_G, _A, ANCHOR_INTERVAL, 1, _D)
    in_spec = pl.BlockSpec((_GB, _A, 1, 1, _D), lambda i: (i, 0, 0, 0, 0))
    out_spec = pl.BlockSpec((_GB, _A, 1, 1, _D), lambda i: (i, 0, 0, 0, 0))
    ko, vo = pl.pallas_call(
        _tc_body,
        grid=(_G // _GB,),
        in_specs=[in_spec, in_spec],
        out_specs=[out_spec, out_spec],
        out_shape=[jax.ShapeDtypeStruct((_G, _A, 1, 1, _D), jnp.float32)] * 2,
    )(k4, v4)
    return (ko.reshape(_B, _H, _A, _D), vo.reshape(_B, _H, _A, _D))
